# baseline (speedup 1.0000x reference)
import functools

import jax
import jax.numpy as jnp
from jax import lax
from jax.experimental import pallas as pl
from jax.experimental.pallas import tpu as pltpu
from jax.experimental.pallas import tpu_sc as plsc

_HID = 32
_CB = 8192
_IN = 256
_N = 8192
_TB = 1024
_NBLK = _N // _TB


_BIG = 2**30
_CHUNK = 2048


def _dist_argmin_body(x_ref, wp_ref, bp_ref, cb_ref, cbt_ref, wb_ref, bb_ref,
                      idx_ref, tab_ref, lsum_ref):
    i = pl.program_id(0)
    h = jnp.dot(x_ref[...], wp_ref[...]) + bp_ref[...][None, :]
    mm = jnp.dot(h, cb_ref[...])
    hnorm = jnp.sum(h * h, axis=1, keepdims=True)
    enorm = jnp.sum(cb_ref[...] * cb_ref[...], axis=0, keepdims=True)
    dist = hnorm - 2.0 * mm + enorm

    tab_ref[...] = lax.dot_general(
        cbt_ref[...], wb_ref[...], (((0,), (0,)), ((), ()))
    ) + bb_ref[...][None, :]

    av = ai = mval = None
    for c in range(_CB // _CHUNK):
        dc = dist[:, c * _CHUNK:(c + 1) * _CHUNK]
        mc = jnp.min(dc, axis=1, keepdims=True)
        jc = (jnp.argmin(dc, axis=1).astype(jnp.int32)
              + jnp.int32(c * _CHUNK))[:, None]
        if av is None:
            av, ai, mval = mc, jc, mc
        else:
            avq = av.astype(jnp.bfloat16).astype(jnp.float32)
            upd = mc < avq
            av = jnp.where(upd, mc, avq)
            ai = jnp.where(upd, jc, ai)
            mval = jnp.where(upd, mc, mval)
    idx_ref[0, 0, :] = ai[:, 0]

    blk = jnp.sum(mval).reshape(1, 1)

    @pl.when(i == 0)
    def _init():
        lsum_ref[...] = blk

    @pl.when(i > 0)
    def _acc():
        lsum_ref[...] += blk


def _make_sc_gather(n_rows, row_dim, chunk=128):
    info = plsc.get_sparse_core_info()
    nc, ns = info.num_cores, info.num_subcores
    nw = nc * ns
    b_per_w = n_rows // nw
    mesh = plsc.VectorSubcoreMesh(core_axis_name="c", subcore_axis_name="s")

    @functools.partial(
        pl.kernel, mesh=mesh,
        out_type=jax.ShapeDtypeStruct((n_rows, row_dim), jnp.float32),
        scratch_types=[
            pltpu.VMEM((b_per_w,), jnp.int32),
            pltpu.VMEM((b_per_w, row_dim), jnp.float32),
            pltpu.SemaphoreType.DMA,
        ],
    )
    def gather_k(idx_hbm, table_hbm, out_hbm, idx_v, rows_v, sem):
        wid = lax.axis_index("s") * nc + lax.axis_index("c")
        base = wid * b_per_w
        pltpu.sync_copy(idx_hbm.at[pl.ds(base, b_per_w)], idx_v)
        handles = [
            pltpu.async_copy(
                table_hbm.at[idx_v.at[pl.ds(j * chunk, chunk)]],
                rows_v.at[pl.ds(j * chunk, chunk)],
                sem,
            )
            for j in range(b_per_w // chunk)
        ]
        for h in handles:
            h.wait()
        pltpu.sync_copy(rows_v, out_hbm.at[pl.ds(base, b_per_w)])

    return gather_k


def kernel(state, W_proj, b_proj, W_back, b_back, codebook):
    bsz, T, _ = state.shape
    x = state.reshape(_N, _IN)

    idx3, table, lsum = pl.pallas_call(
        _dist_argmin_body,
        grid=(_NBLK,),
        in_specs=[
            pl.BlockSpec((_TB, _IN), lambda i: (i, 0)),
            pl.BlockSpec((_IN, _HID), lambda i: (0, 0)),
            pl.BlockSpec((_HID,), lambda i: (0,)),
            pl.BlockSpec((_HID, _CB), lambda i: (0, 0)),
            pl.BlockSpec((_HID, _TB), lambda i: (0, i)),
            pl.BlockSpec((_HID, _IN), lambda i: (0, 0)),
            pl.BlockSpec((_IN,), lambda i: (0,)),
        ],
        out_specs=[
            pl.BlockSpec((1, 1, _TB), lambda i: (i, 0, 0)),
            pl.BlockSpec((_TB, _IN), lambda i: (i, 0)),
            pl.BlockSpec((1, 1), lambda i: (0, 0)),
        ],
        out_shape=[
            jax.ShapeDtypeStruct((_NBLK, 1, _TB), jnp.int32),
            jax.ShapeDtypeStruct((_CB, _IN), jnp.float32),
            jax.ShapeDtypeStruct((1, 1), jnp.float32),
        ],
    )(x, W_proj, b_proj, codebook, codebook, W_back, b_back)

    idx = idx3.reshape(_N)
    out_rows = _make_sc_gather(_N, _IN)(idx, table)

    out = out_rows.reshape(bsz, T, _IN)
    cb_loss = lsum[0, 0] / jnp.float32(_N * _HID)
    att_scores = jnp.zeros((1, 1, 2), dtype=jnp.float32)
    return out, cb_loss, att_scores


# speedup vs baseline: 1.0356x; 1.0356x over previous
import functools

import jax
import jax.numpy as jnp
from jax import lax
from jax.experimental import pallas as pl
from jax.experimental.pallas import tpu as pltpu
from jax.experimental.pallas import tpu_sc as plsc

_HID = 32
_CB = 8192
_IN = 256
_N = 8192
_TB = 512
_NBLK = _N // _TB


_BIG = 2**30
_CHUNK = 2048


def _dist_argmin_body(x_ref, wp_ref, bp_ref, cb_ref, cbt_ref, wb_ref, bb_ref,
                      idx_ref, tab_ref, lsum_ref, dist_ref):
    i = pl.program_id(0)
    h = jnp.dot(x_ref[...], wp_ref[...]) + bp_ref[...][None, :]
    mm = jnp.dot(h, cb_ref[...])
    hnorm = jnp.sum(h * h, axis=1, keepdims=True)
    enorm = jnp.sum(cb_ref[...] * cb_ref[...], axis=0, keepdims=True)
    dist_ref[...] = hnorm - 2.0 * mm + enorm
    dist = dist_ref[...]

    tab_ref[...] = lax.dot_general(
        cbt_ref[...], wb_ref[...], (((0,), (0,)), ((), ()))
    ) + bb_ref[...][None, :]

    iota = lax.broadcasted_iota(jnp.int32, (dist.shape[0], _CHUNK), 1)
    av = ai = mval = None
    for c in range(_CB // _CHUNK):
        dc = dist[:, c * _CHUNK:(c + 1) * _CHUNK]
        mc = jnp.min(dc, axis=1, keepdims=True)
        jc = (jnp.min(jnp.where(dc == mc, iota, _BIG), axis=1, keepdims=True)
              + jnp.int32(c * _CHUNK))
        if av is None:
            av, ai, mval = mc, jc, mc
        else:
            avq = av.astype(jnp.bfloat16).astype(jnp.float32)
            upd = mc < avq
            av = jnp.where(upd, mc, avq)
            ai = jnp.where(upd, jc, ai)
            mval = jnp.where(upd, mc, mval)
    idx_ref[0, 0, :] = ai[:, 0]

    blk = jnp.sum(mval).reshape(1, 1)

    @pl.when(i == 0)
    def _init():
        lsum_ref[...] = blk

    @pl.when(i > 0)
    def _acc():
        lsum_ref[...] += blk


def _make_sc_gather(n_rows, row_dim, chunk=128):
    info = plsc.get_sparse_core_info()
    nc, ns = info.num_cores, info.num_subcores
    nw = nc * ns
    b_per_w = n_rows // nw
    mesh = plsc.VectorSubcoreMesh(core_axis_name="c", subcore_axis_name="s")

    @functools.partial(
        pl.kernel, mesh=mesh,
        out_type=jax.ShapeDtypeStruct((n_rows, row_dim), jnp.float32),
        scratch_types=[
            pltpu.VMEM((b_per_w,), jnp.int32),
            pltpu.VMEM((b_per_w, row_dim), jnp.float32),
            pltpu.SemaphoreType.DMA,
        ],
    )
    def gather_k(idx_hbm, table_hbm, out_hbm, idx_v, rows_v, sem):
        wid = lax.axis_index("s") * nc + lax.axis_index("c")
        base = wid * b_per_w
        pltpu.sync_copy(idx_hbm.at[pl.ds(base, b_per_w)], idx_v)
        handles = [
            pltpu.async_copy(
                table_hbm.at[idx_v.at[pl.ds(j * chunk, chunk)]],
                rows_v.at[pl.ds(j * chunk, chunk)],
                sem,
            )
            for j in range(b_per_w // chunk)
        ]
        for h in handles:
            h.wait()
        pltpu.sync_copy(rows_v, out_hbm.at[pl.ds(base, b_per_w)])

    return gather_k


def kernel(state, W_proj, b_proj, W_back, b_back, codebook):
    bsz, T, _ = state.shape
    x = state.reshape(_N, _IN)

    idx3, table, lsum = pl.pallas_call(
        _dist_argmin_body,
        grid=(_NBLK,),
        in_specs=[
            pl.BlockSpec((_TB, _IN), lambda i: (i, 0)),
            pl.BlockSpec((_IN, _HID), lambda i: (0, 0)),
            pl.BlockSpec((_HID,), lambda i: (0,)),
            pl.BlockSpec((_HID, _CB), lambda i: (0, 0)),
            pl.BlockSpec((_HID, _TB), lambda i: (0, i)),
            pl.BlockSpec((_HID, _IN), lambda i: (0, 0)),
            pl.BlockSpec((_IN,), lambda i: (0,)),
        ],
        out_specs=[
            pl.BlockSpec((1, 1, _TB), lambda i: (i, 0, 0)),
            pl.BlockSpec((_TB, _IN), lambda i: (i, 0)),
            pl.BlockSpec((1, 1), lambda i: (0, 0)),
        ],
        out_shape=[
            jax.ShapeDtypeStruct((_NBLK, 1, _TB), jnp.int32),
            jax.ShapeDtypeStruct((_CB, _IN), jnp.float32),
            jax.ShapeDtypeStruct((1, 1), jnp.float32),
        ],
        scratch_shapes=[pltpu.VMEM((_TB, _CB), jnp.float32)],
    )(x, W_proj, b_proj, codebook, codebook, W_back, b_back)

    idx = idx3.reshape(_N)
    out_rows = _make_sc_gather(_N, _IN)(idx, table)

    out = out_rows.reshape(bsz, T, _IN)
    cb_loss = lsum[0, 0] / jnp.float32(_N * _HID)
    att_scores = jnp.zeros((1, 1, 2), dtype=jnp.float32)
    return out, cb_loss, att_scores


# speedup vs baseline: 1.0364x; 1.0007x over previous
import functools

import jax
import jax.numpy as jnp
from jax import lax
from jax.experimental import pallas as pl
from jax.experimental.pallas import tpu as pltpu
from jax.experimental.pallas import tpu_sc as plsc

_HID = 32
_CB = 8192
_IN = 256
_N = 8192
_TB = 512
_NBLK = _N // _TB


_BIG = 2**30
_CHUNK = 2048


def _dist_argmin_body(x_ref, wp_ref, bp_ref, cb_ref, cbt_ref, wb_ref, bb_ref,
                      idx_ref, tab_ref, lsum_ref):
    i = pl.program_id(0)
    h = jnp.dot(x_ref[...], wp_ref[...]) + bp_ref[...][None, :]
    mm = jnp.dot(h, cb_ref[...])
    hnorm = jnp.sum(h * h, axis=1, keepdims=True)
    enorm = jnp.sum(cb_ref[...] * cb_ref[...], axis=0, keepdims=True)
    dist = hnorm - 2.0 * mm + enorm

    tab_ref[...] = lax.dot_general(
        cbt_ref[...], wb_ref[...], (((0,), (0,)), ((), ()))
    ) + bb_ref[...][None, :]

    iota = lax.broadcasted_iota(jnp.int32, (dist.shape[0], _CHUNK), 1)
    av = ai = mval = None
    for c in range(_CB // _CHUNK):
        dc = dist[:, c * _CHUNK:(c + 1) * _CHUNK]
        mc = jnp.min(dc, axis=1, keepdims=True)
        jc = (jnp.min(jnp.where(dc == mc, iota, _BIG), axis=1, keepdims=True)
              + jnp.int32(c * _CHUNK))
        if av is None:
            av, ai, mval = mc, jc, mc
        else:
            avq = av.astype(jnp.bfloat16).astype(jnp.float32)
            upd = mc < avq
            av = jnp.where(upd, mc, avq)
            ai = jnp.where(upd, jc, ai)
            mval = jnp.where(upd, mc, mval)
    idx_ref[0, 0, :] = ai[:, 0]

    blk = jnp.sum(mval).reshape(1, 1)

    @pl.when(i == 0)
    def _init():
        lsum_ref[...] = blk

    @pl.when(i > 0)
    def _acc():
        lsum_ref[...] += blk


def _make_sc_gather(n_rows, row_dim, chunk=128):
    info = plsc.get_sparse_core_info()
    nc, ns = info.num_cores, info.num_subcores
    nw = nc * ns
    b_per_w = n_rows // nw
    mesh = plsc.VectorSubcoreMesh(core_axis_name="c", subcore_axis_name="s")

    @functools.partial(
        pl.kernel, mesh=mesh,
        out_type=jax.ShapeDtypeStruct((n_rows, row_dim), jnp.float32),
        scratch_types=[
            pltpu.VMEM((b_per_w,), jnp.int32),
            pltpu.VMEM((b_per_w, row_dim), jnp.float32),
            pltpu.SemaphoreType.DMA,
        ],
    )
    def gather_k(idx_hbm, table_hbm, out_hbm, idx_v, rows_v, sem):
        wid = lax.axis_index("s") * nc + lax.axis_index("c")
        base = wid * b_per_w
        pltpu.sync_copy(idx_hbm.at[pl.ds(base, b_per_w)], idx_v)
        handles = [
            pltpu.async_copy(
                table_hbm.at[idx_v.at[pl.ds(j * chunk, chunk)]],
                rows_v.at[pl.ds(j * chunk, chunk)],
                sem,
            )
            for j in range(b_per_w // chunk)
        ]
        for h in handles:
            h.wait()
        pltpu.sync_copy(rows_v, out_hbm.at[pl.ds(base, b_per_w)])

    return gather_k


def kernel(state, W_proj, b_proj, W_back, b_back, codebook):
    bsz, T, _ = state.shape
    x = state.reshape(_N, _IN)

    idx3, table, lsum = pl.pallas_call(
        _dist_argmin_body,
        grid=(_NBLK,),
        in_specs=[
            pl.BlockSpec((_TB, _IN), lambda i: (i, 0)),
            pl.BlockSpec((_IN, _HID), lambda i: (0, 0)),
            pl.BlockSpec((_HID,), lambda i: (0,)),
            pl.BlockSpec((_HID, _CB), lambda i: (0, 0)),
            pl.BlockSpec((_HID, _TB), lambda i: (0, i)),
            pl.BlockSpec((_HID, _IN), lambda i: (0, 0)),
            pl.BlockSpec((_IN,), lambda i: (0,)),
        ],
        out_specs=[
            pl.BlockSpec((1, 1, _TB), lambda i: (i, 0, 0)),
            pl.BlockSpec((_TB, _IN), lambda i: (i, 0)),
            pl.BlockSpec((1, 1), lambda i: (0, 0)),
        ],
        out_shape=[
            jax.ShapeDtypeStruct((_NBLK, 1, _TB), jnp.int32),
            jax.ShapeDtypeStruct((_CB, _IN), jnp.float32),
            jax.ShapeDtypeStruct((1, 1), jnp.float32),
        ],
    )(x, W_proj, b_proj, codebook, codebook, W_back, b_back)

    idx = idx3.reshape(_N)
    out_rows = _make_sc_gather(_N, _IN)(idx, table)

    out = out_rows.reshape(bsz, T, _IN)
    cb_loss = lsum[0, 0] / jnp.float32(_N * _HID)
    att_scores = jnp.zeros((1, 1, 2), dtype=jnp.float32)
    return out, cb_loss, att_scores
